# Initial kernel scaffold; baseline (speedup 1.0000x reference)
import jax
import jax.numpy as jnp
from jax import lax
from jax.experimental import pallas as pl
from jax.experimental.pallas import tpu as pltpu
from jax.experimental.pallas import tpu_sc as plsc

N_NODES = 10000
N_EDGES = 320000
D_FEAT = 128
OUT_F = 4

NUM_CORES = 2
NUM_SUBCORES = 16
NW = NUM_CORES * NUM_SUBCORES
EPW = N_EDGES // NW
CHUNK = 2000
GROUPS = CHUNK // 16


def _mm_body(x_ref, w_ref, o_ref):
    xb = x_ref[:]
    w = w_ref[:]
    dn = (((1,), (1,)), ((), ()))
    y = lax.dot_general(xb, w[:, :D_FEAT], dn, preferred_element_type=jnp.float32)
    z = lax.dot_general(xb, w[:, D_FEAT:], dn, preferred_element_type=jnp.float32)
    o_ref[:] = jnp.concatenate([y, z], axis=1)


_mm_call = pl.pallas_call(
    _mm_body,
    out_shape=jax.ShapeDtypeStruct((N_NODES, 2 * OUT_F), jnp.float32),
)


def _sc_body(table_hbm, rows_hbm, cols_hbm, out_hbm, table_v, rows_v, cols_v, outst_v):
    wid = lax.axis_index("s") * NUM_CORES + lax.axis_index("c")
    pltpu.sync_copy(table_hbm, table_v)

    def chunk_body(ch, carry):
        base = wid * EPW + ch * CHUNK
        pltpu.sync_copy(rows_hbm.at[pl.ds(base, CHUNK)], rows_v)
        pltpu.sync_copy(cols_hbm.at[pl.ds(base, CHUNK)], cols_v)

        def group_body(g, carry2):
            r = rows_v[pl.ds(g * 16, 16)]
            c = cols_v[pl.ds(g * 16, 16)]
            rb = r * 8
            cb = c * 8 + 4
            obase = lax.iota(jnp.int32, 16) * OUT_F + g * (16 * OUT_F)
            for j in range(OUT_F):
                yj = plsc.load_gather(table_v, [rb + j])
                zj = plsc.load_gather(table_v, [cb + j])
                s = yj + zj
                t = 1.0 - 2.0 / (jnp.exp(2.0 * s) + 1.0)
                plsc.store_scatter(outst_v, [obase + j], t)
            return carry2

        lax.fori_loop(0, GROUPS, group_body, 0)
        pltpu.sync_copy(outst_v, out_hbm.at[pl.ds(base * OUT_F, CHUNK * OUT_F)])
        return carry

    lax.fori_loop(0, EPW // CHUNK, chunk_body, 0)


_sc_call = pl.kernel(
    _sc_body,
    out_type=jax.ShapeDtypeStruct((N_EDGES * OUT_F,), jnp.float32),
    mesh=plsc.VectorSubcoreMesh(core_axis_name="c", subcore_axis_name="s"),
    scratch_types=[
        pltpu.VMEM((N_NODES * 2 * OUT_F,), jnp.float32),
        pltpu.VMEM((CHUNK,), jnp.int32),
        pltpu.VMEM((CHUNK,), jnp.int32),
        pltpu.VMEM((CHUNK * OUT_F,), jnp.float32),
    ],
)


@jax.jit
def kernel(x, edge_index, W):
    table = _mm_call(x, W)
    rows = edge_index[0].astype(jnp.int32)
    cols = edge_index[1].astype(jnp.int32)
    out_flat = _sc_call(table.reshape(-1), rows, cols)
    return out_flat.reshape(N_EDGES, 2, 2)


# speedup vs baseline: 1.4054x; 1.4054x over previous
import jax
import jax.numpy as jnp
from jax import lax
from jax.experimental import pallas as pl
from jax.experimental.pallas import tpu as pltpu
from jax.experimental.pallas import tpu_sc as plsc

N_NODES = 10000
N_EDGES = 320000
D_FEAT = 128
OUT_F = 4

NUM_CORES = 2
NUM_SUBCORES = 16
NW = NUM_CORES * NUM_SUBCORES
EPW = N_EDGES // NW
CHUNK = 2000
GROUPS = CHUNK // 16


def _mm_body(x_ref, w_ref, o_ref):
    xb = x_ref[:]
    w = w_ref[:]
    dn = (((1,), (1,)), ((), ()))
    y = lax.dot_general(xb, w[:, :D_FEAT], dn, preferred_element_type=jnp.float32)
    z = lax.dot_general(xb, w[:, D_FEAT:], dn, preferred_element_type=jnp.float32)
    o_ref[:] = jnp.concatenate([y, z], axis=1)


_mm_call = pl.pallas_call(
    _mm_body,
    out_shape=jax.ShapeDtypeStruct((N_NODES, 2 * OUT_F), jnp.float32),
)


def _sc_body(table_hbm, rows_hbm, cols_hbm, out_hbm, table_v, rows_v, cols_v, outst_v):
    wid = lax.axis_index("s") * NUM_CORES + lax.axis_index("c")
    pltpu.sync_copy(table_hbm, table_v)

    def chunk_body(ch, carry):
        base = wid * EPW + ch * CHUNK
        pltpu.sync_copy(rows_hbm.at[pl.ds(base, CHUNK)], rows_v)
        pltpu.sync_copy(cols_hbm.at[pl.ds(base, CHUNK)], cols_v)

        def group_body(g, carry2):
            r = rows_v[pl.ds(g * 16, 16)]
            c = cols_v[pl.ds(g * 16, 16)]
            rb = r * 8
            cb = c * 8 + 4
            obase = lax.iota(jnp.int32, 16) * OUT_F + g * (16 * OUT_F)
            for j in range(OUT_F):
                yj = plsc.load_gather(table_v, [rb + j])
                zj = plsc.load_gather(table_v, [cb + j])
                s = yj + zj
                t = 1.0 - 2.0 / (jnp.exp(2.0 * s) + 1.0)
                plsc.store_scatter(outst_v, [obase + j], t)
            return carry2

        lax.fori_loop(0, GROUPS, group_body, 0)
        pltpu.sync_copy(outst_v, out_hbm.at[pl.ds(base * OUT_F, CHUNK * OUT_F)])
        return carry

    lax.fori_loop(0, EPW // CHUNK, chunk_body, 0)


_sc_call = pl.kernel(
    _sc_body,
    out_type=jax.ShapeDtypeStruct((N_EDGES * OUT_F,), jnp.float32),
    mesh=plsc.VectorSubcoreMesh(core_axis_name="c", subcore_axis_name="s"),
    compiler_params=pltpu.CompilerParams(needs_layout_passes=False),
    scratch_types=[
        pltpu.VMEM((N_NODES * 2 * OUT_F,), jnp.float32),
        pltpu.VMEM((CHUNK,), jnp.int32),
        pltpu.VMEM((CHUNK,), jnp.int32),
        pltpu.VMEM((CHUNK * OUT_F,), jnp.float32),
    ],
)


@jax.jit
def kernel(x, edge_index, W):
    table = _mm_call(x, W)
    rows = edge_index[0].astype(jnp.int32)
    cols = edge_index[1].astype(jnp.int32)
    out_flat = _sc_call(table.reshape(-1), rows, cols)
    return out_flat.reshape(N_EDGES, 2, 2)
